# baseline (speedup 1.0000x reference)
import functools

import jax
import jax.numpy as jnp
from jax import lax
from jax.experimental import pallas as pl
from jax.experimental.pallas import tpu as pltpu
from jax.experimental.pallas import tpu_sc as plsc

_SC_BATCHES = 2
_TC_BLK = 256


def _make_sc_broadcast(batch, seq_len, d_model, dtype):
    info = plsc.get_sparse_core_info()
    num_workers = info.num_cores * info.num_subcores
    rows_per_worker = seq_len // num_workers
    chunk = min(64, rows_per_worker)
    num_chunks = rows_per_worker // chunk

    mesh = plsc.VectorSubcoreMesh(core_axis_name="c", subcore_axis_name="s")

    @functools.partial(
        pl.kernel,
        mesh=mesh,
        out_type=jax.ShapeDtypeStruct((batch, seq_len, d_model), dtype),
        scratch_types=[
            pltpu.VMEM((chunk, d_model), dtype),
            pltpu.SemaphoreType.DMA,
        ],
    )
    def sc_broadcast(table_hbm, out_hbm, buf, sem):
        wid = lax.axis_index("s") * info.num_cores + lax.axis_index("c")
        base = wid * rows_per_worker

        def body(i, carry):
            r0 = base + i * chunk
            pltpu.sync_copy(table_hbm.at[pl.ds(r0, chunk)], buf)
            for b in range(batch):
                pltpu.sync_copy(buf, out_hbm.at[b, pl.ds(r0, chunk)])
            return carry

        lax.fori_loop(0, num_chunks, body, 0)

    return sc_broadcast


def _tc_body(t_ref, o_ref):
    o_ref[...] = jnp.broadcast_to(t_ref[...][None], o_ref.shape)


def _tc_broadcast(batch, seq_len, d_model, dtype, table):
    return pl.pallas_call(
        _tc_body,
        grid=(seq_len // _TC_BLK,),
        in_specs=[pl.BlockSpec((_TC_BLK, d_model), lambda i: (i, 0))],
        out_specs=pl.BlockSpec((batch, _TC_BLK, d_model), lambda i: (0, i, 0)),
        out_shape=jax.ShapeDtypeStruct((batch, seq_len, d_model), dtype),
    )(table)


def kernel(x, table):
    batch, seq_len, d_model = x.shape
    sc_out = _make_sc_broadcast(_SC_BATCHES, seq_len, d_model, table.dtype)(table)
    tc_out = _tc_broadcast(batch - _SC_BATCHES, seq_len, d_model, table.dtype, table)
    return jnp.concatenate([sc_out, tc_out], axis=0)


# speedup vs baseline: 1.9875x; 1.9875x over previous
import functools

import jax
import jax.numpy as jnp
from jax import lax
from jax.experimental import pallas as pl
from jax.experimental.pallas import tpu as pltpu
from jax.experimental.pallas import tpu_sc as plsc


def _make_sc_broadcast(batch, seq_len, d_model, dtype):
    info = plsc.get_sparse_core_info()
    num_workers = info.num_cores * info.num_subcores
    rows_per_worker = seq_len // num_workers
    chunk = min(64, rows_per_worker)
    num_chunks = rows_per_worker // chunk
    ns = info.num_subcores

    mesh = plsc.VectorSubcoreMesh(core_axis_name="c", subcore_axis_name="s")

    @functools.partial(
        pl.kernel,
        mesh=mesh,
        out_type=jax.ShapeDtypeStruct((batch, seq_len, d_model), dtype),
        scratch_types=[
            pltpu.VMEM((chunk, d_model), dtype),
            pltpu.VMEM_SHARED((ns * chunk, d_model), dtype),
            pltpu.SemaphoreType.DMA,
            pltpu.SemaphoreType.DMA,
            pltpu.SemaphoreType.DMA,
            pltpu.SemaphoreType.DMA,
        ],
    )
    def sc_broadcast(table_hbm, out_hbm, tbuf, sbuf, rs, rs2, ws, ws2):
        cid = lax.axis_index("c")
        sid = lax.axis_index("s")
        wid = sid * info.num_cores + cid
        base = wid * rows_per_worker
        my_sbuf = sbuf.at[pl.ds(sid * chunk, chunk)]

        def body(i, carry):
            r0 = base + i * chunk
            src = table_hbm.at[pl.ds(r0, chunk)]
            ra = pltpu.async_copy(src, tbuf, rs)
            rb = pltpu.async_copy(src, my_sbuf, rs2)
            ra.wait()
            stream_writes = [
                pltpu.async_copy(tbuf, out_hbm.at[b, pl.ds(r0, chunk)], ws)
                for b in range(batch - 1)
            ]
            rb.wait()
            wb = pltpu.async_copy(
                my_sbuf, out_hbm.at[batch - 1, pl.ds(r0, chunk)], ws2
            )
            for h in stream_writes:
                h.wait()
            wb.wait()
            return carry

        lax.fori_loop(0, num_chunks, body, 0)

    return sc_broadcast


def kernel(x, table):
    batch, seq_len, d_model = x.shape
    fn = _make_sc_broadcast(batch, seq_len, d_model, table.dtype)
    return fn(table)


# speedup vs baseline: 2.2894x; 1.1519x over previous
import functools

import jax
import jax.numpy as jnp
from jax import lax
from jax.experimental import pallas as pl
from jax.experimental.pallas import tpu as pltpu
from jax.experimental.pallas import tpu_sc as plsc


def _make_sc_broadcast(batch, seq_len, d_model, dtype):
    info = plsc.get_sparse_core_info()
    num_workers = info.num_cores * info.num_subcores
    rows_per_worker = seq_len // num_workers
    chunk = min(64, rows_per_worker)
    num_chunks = rows_per_worker // chunk

    mesh = plsc.VectorSubcoreMesh(core_axis_name="c", subcore_axis_name="s")

    @functools.partial(
        pl.kernel,
        mesh=mesh,
        out_type=jax.ShapeDtypeStruct((batch, seq_len, d_model), dtype),
        scratch_types=[
            pltpu.VMEM((chunk, d_model), dtype),
            pltpu.SemaphoreType.DMA,
            pltpu.SemaphoreType.DMA,
        ],
    )
    def sc_broadcast(table_hbm, out_hbm, buf, rsem, wsem):
        wid = lax.axis_index("s") * info.num_cores + lax.axis_index("c")
        base = wid * rows_per_worker

        def body(i, carry):
            r0 = base + i * chunk
            pltpu.async_copy(table_hbm.at[pl.ds(r0, chunk)], buf, rsem).wait()
            writes = [
                pltpu.async_copy(buf, out_hbm.at[b, pl.ds(r0, chunk)], wsem)
                for b in range(batch)
            ]
            for h in writes:
                h.wait()
            return carry

        lax.fori_loop(0, num_chunks, body, 0)

    return sc_broadcast


def kernel(x, table):
    batch, seq_len, d_model = x.shape
    fn = _make_sc_broadcast(batch, seq_len, d_model, table.dtype)
    return fn(table)
